# Initial kernel scaffold; baseline (speedup 1.0000x reference)
import jax
import jax.numpy as jnp
from jax.experimental import pallas as pl


def kernel(obs_bar_reward, barcode_tensor, barcode_id, h, c, dnd_keys, dnd_vals, key_id_map, W_i2h, b_i2h, W_h2h, b_h2h, W_a2c_h, b_a2c_h, W_pi, b_pi, W_v, b_v):
    raise NotImplementedError("write your pallas kernel here")


# speedup vs baseline: 1.4890x; 1.4890x over previous
import functools

import jax
import jax.numpy as jnp
from jax import lax
from jax.experimental import pallas as pl
from jax.experimental.pallas import tpu as pltpu
from jax.experimental.pallas import tpu_sc as plsc

N_GATES = 4
B = 128
D_IN = 512
D_H = 512
D_A2C = 256
D_OUT = 10
DICT_LEN = 100000

BLK = 2000
NBLK = DICT_LEN // BLK
NEG_BIG = -1e30


def _sim_body(q_ref, k_ref, best_ref, maxv, bestv):
    i = pl.program_id(0)
    q = q_ref[...]
    qn = q / (jnp.sqrt(jnp.sum(q * q, axis=1, keepdims=True)) + 1e-8)
    k = k_ref[...]
    kn = k / (jnp.sqrt(jnp.sum(k * k, axis=1, keepdims=True)) + 1e-8)
    sims = lax.dot_general(qn, kn, (((1,), (1,)), ((), ())),
                           preferred_element_type=jnp.float32)
    bmax = jnp.max(sims, axis=1, keepdims=True)
    iota = lax.broadcasted_iota(jnp.int32, (B, BLK), 1)
    bidx = jnp.min(jnp.where(sims == bmax, iota, DICT_LEN),
                   axis=1, keepdims=True) + i * BLK

    @pl.when(i == 0)
    def _():
        maxv[...] = bmax
        bestv[...] = bidx

    @pl.when(i > 0)
    def _():
        upd = bmax > maxv[...]
        maxv[...] = jnp.where(upd, bmax, maxv[...])
        bestv[...] = jnp.where(upd, bidx, bestv[...])

    @pl.when(i == NBLK - 1)
    def _():
        best_ref[...] = bestv[...]


_sim_call = pl.pallas_call(
    _sim_body,
    grid=(NBLK,),
    in_specs=[
        pl.BlockSpec((B, D_IN), lambda i: (0, 0)),
        pl.BlockSpec((BLK, D_IN), lambda i: (i, 0)),
    ],
    out_specs=pl.BlockSpec((B, 1), lambda i: (0, 0)),
    out_shape=jax.ShapeDtypeStruct((B, 1), jnp.int32),
    scratch_shapes=[
        pltpu.VMEM((B, 1), jnp.float32),
        pltpu.VMEM((B, 1), jnp.int32),
    ],
    compiler_params=pltpu.CompilerParams(
        dimension_semantics=("arbitrary",),
    ),
)


_NC = 2
_NW_USED = 16
_BPW = B // _NW_USED


def _gather_body(idx_hbm, vals_hbm, kim_hbm, mem_out, bc_out,
                 idx_v, rows_v, bc_v, sem0, sem1):
    wid = lax.axis_index("s") * _NC + lax.axis_index("c")

    @pl.when(wid < _NW_USED)
    def _():
        base = wid * _BPW
        pltpu.sync_copy(idx_hbm.at[pl.ds(base, _BPW)], idx_v)
        pltpu.async_copy(vals_hbm.at[idx_v], rows_v, sem0).wait()
        pltpu.async_copy(kim_hbm.at[idx_v], bc_v, sem1).wait()
        pltpu.sync_copy(rows_v, mem_out.at[pl.ds(base, _BPW)])
        pltpu.sync_copy(bc_v, bc_out.at[pl.ds(base, _BPW)])


@functools.lru_cache(maxsize=1)
def _sc_gather_call():
    return functools.partial(
        pl.kernel,
        mesh=plsc.VectorSubcoreMesh(core_axis_name="c", subcore_axis_name="s"),
        out_type=[
            jax.ShapeDtypeStruct((B, D_H), jnp.float32),
            jax.ShapeDtypeStruct((B,), jnp.int32),
        ],
        scratch_types=[
            pltpu.VMEM((_BPW,), jnp.int32),
            pltpu.VMEM((_BPW, D_H), jnp.float32),
            pltpu.VMEM((_BPW,), jnp.int32),
            pltpu.SemaphoreType.DMA,
            pltpu.SemaphoreType.DMA,
        ],
    )(_gather_body)


def _head_body(x_ref, h_ref, c_ref, mem_ref, wi_ref, wh_ref, bl_ref,
               wa_ref, ba_ref, wpv_ref, bpv_ref,
               a_ref, p_ref, v_ref, e_ref, ho_ref, co_ref):
    x = x_ref[...]
    hh = h_ref[...]
    cc = c_ref[...]
    preact = (jnp.dot(x, wi_ref[...], preferred_element_type=jnp.float32)
              + jnp.dot(hh, wh_ref[...], preferred_element_type=jnp.float32)
              + bl_ref[...])
    gates = jax.nn.sigmoid(preact[:, :N_GATES * D_H])
    f_t = gates[:, :D_H]
    i_t = gates[:, D_H:2 * D_H]
    o_t = gates[:, 2 * D_H:3 * D_H]
    r_t = gates[:, 3 * D_H:4 * D_H]
    c_tilde = jnp.tanh(preact[:, N_GATES * D_H:])
    m_t = jnp.tanh(mem_ref[...])
    c_t = f_t * cc + i_t * c_tilde + r_t * m_t
    h_t = o_t * jnp.tanh(c_t)
    a_hid = jnp.maximum(
        jnp.dot(h_t, wa_ref[...], preferred_element_type=jnp.float32)
        + ba_ref[...], 0.0)
    logits = (jnp.dot(a_hid, wpv_ref[...], preferred_element_type=jnp.float32)
              + bpv_ref[...])
    v_ref[...] = logits[:, D_OUT:D_OUT + 1]
    col = lax.broadcasted_iota(jnp.int32, (B, 128), 1)
    masked = jnp.where(col < D_OUT, logits, NEG_BIG)
    m = jnp.max(masked, axis=1, keepdims=True)
    e = jnp.exp(masked - m)
    pi = e / jnp.sum(e, axis=1, keepdims=True)
    pmax = jnp.max(pi, axis=1, keepdims=True)
    a_ref[...] = jnp.min(jnp.where(pi == pmax, col, 128), axis=1, keepdims=True)
    p_ref[...] = jnp.log(pmax + 1e-12)
    e_ref[...] = -jnp.sum(pi * jnp.log(pi + 1e-12), axis=1, keepdims=True)
    ho_ref[...] = h_t
    co_ref[...] = c_t


_head_call = pl.pallas_call(
    _head_body,
    out_shape=(
        jax.ShapeDtypeStruct((B, 1), jnp.int32),
        jax.ShapeDtypeStruct((B, 1), jnp.float32),
        jax.ShapeDtypeStruct((B, 1), jnp.float32),
        jax.ShapeDtypeStruct((B, 1), jnp.float32),
        jax.ShapeDtypeStruct((B, D_H), jnp.float32),
        jax.ShapeDtypeStruct((B, D_H), jnp.float32),
    ),
)


def kernel(obs_bar_reward, barcode_tensor, barcode_id, h, c, dnd_keys,
           dnd_vals, key_id_map, W_i2h, b_i2h, W_h2h, b_h2h,
           W_a2c_h, b_a2c_h, W_pi, b_pi, W_v, b_v):
    best = _sim_call(barcode_tensor, dnd_keys).reshape(B)
    mem, predicted_barcode = _sc_gather_call()(best, dnd_vals, key_id_map)

    wpv = jnp.zeros((D_A2C, 128), jnp.float32)
    wpv = wpv.at[:, :D_OUT].set(W_pi.T).at[:, D_OUT].set(W_v[0])
    bpv = jnp.zeros((128,), jnp.float32)
    bpv = bpv.at[:D_OUT].set(b_pi).at[D_OUT].set(b_v[0])

    a_t, prob_a_t, v_t, entropy, h_t, c_t = _head_call(
        obs_bar_reward, h, c, mem,
        W_i2h.T, W_h2h.T, (b_i2h + b_h2h).reshape(1, -1),
        W_a2c_h.T, b_a2c_h.reshape(1, -1),
        wpv, bpv.reshape(1, -1))
    return (a_t.reshape(B), predicted_barcode, prob_a_t.reshape(B), v_t,
            entropy.reshape(B), h_t, c_t)


# speedup vs baseline: 1.6098x; 1.0811x over previous
import functools

import jax
import jax.numpy as jnp
from jax import lax
from jax.experimental import pallas as pl
from jax.experimental.pallas import tpu as pltpu
from jax.experimental.pallas import tpu_sc as plsc

N_GATES = 4
B = 128
D_IN = 512
D_H = 512
D_A2C = 256
D_OUT = 10
DICT_LEN = 100000

BLK = 2000
NBLK = DICT_LEN // BLK
NEG_BIG = -1e30


def _sim_body(q_ref, k_ref, best_ref, maxv, bestv):
    i = pl.program_id(0)
    q = q_ref[...]
    qn = q / (jnp.sqrt(jnp.sum(q * q, axis=1, keepdims=True)) + 1e-8)
    k = k_ref[...]
    ksq = k * k
    ones_row = jnp.ones((1, D_IN), jnp.float32)
    nrow = lax.dot_general(ones_row, ksq, (((1,), (1,)), ((), ())),
                           preferred_element_type=jnp.float32)
    rscale = 1.0 / (jnp.sqrt(nrow) + 1e-8)
    raw = lax.dot_general(qn, k, (((1,), (1,)), ((), ())),
                          preferred_element_type=jnp.float32)
    sims = raw * rscale
    bmax = jnp.max(sims, axis=1, keepdims=True)
    iota = lax.broadcasted_iota(jnp.int32, (B, BLK), 1)
    bidx = jnp.min(jnp.where(sims == bmax, iota, DICT_LEN),
                   axis=1, keepdims=True) + i * BLK

    @pl.when(i == 0)
    def _():
        maxv[...] = bmax
        bestv[...] = bidx

    @pl.when(i > 0)
    def _():
        upd = bmax > maxv[...]
        maxv[...] = jnp.where(upd, bmax, maxv[...])
        bestv[...] = jnp.where(upd, bidx, bestv[...])

    @pl.when(i == NBLK - 1)
    def _():
        best_ref[...] = bestv[...]


_sim_call = pl.pallas_call(
    _sim_body,
    grid=(NBLK,),
    in_specs=[
        pl.BlockSpec((B, D_IN), lambda i: (0, 0)),
        pl.BlockSpec((BLK, D_IN), lambda i: (i, 0)),
    ],
    out_specs=pl.BlockSpec((B, 1), lambda i: (0, 0)),
    out_shape=jax.ShapeDtypeStruct((B, 1), jnp.int32),
    scratch_shapes=[
        pltpu.VMEM((B, 1), jnp.float32),
        pltpu.VMEM((B, 1), jnp.int32),
    ],
    compiler_params=pltpu.CompilerParams(
        dimension_semantics=("arbitrary",),
    ),
)


_NC = 2
_NW_USED = 16
_BPW = B // _NW_USED


def _gather_body(idx_hbm, vals_hbm, kim_hbm, mem_out, bc_out,
                 idx_v, rows_v, bc_v, sem0, sem1):
    wid = lax.axis_index("s") * _NC + lax.axis_index("c")

    @pl.when(wid < _NW_USED)
    def _():
        base = wid * _BPW
        pltpu.sync_copy(idx_hbm.at[pl.ds(base, _BPW)], idx_v)
        cp0 = pltpu.async_copy(vals_hbm.at[idx_v], rows_v, sem0)
        cp1 = pltpu.async_copy(kim_hbm.at[idx_v], bc_v, sem1)
        cp0.wait()
        cp1.wait()
        pltpu.sync_copy(rows_v, mem_out.at[pl.ds(base, _BPW)])
        pltpu.sync_copy(bc_v, bc_out.at[pl.ds(base, _BPW)])


@functools.lru_cache(maxsize=1)
def _sc_gather_call():
    return functools.partial(
        pl.kernel,
        mesh=plsc.VectorSubcoreMesh(core_axis_name="c", subcore_axis_name="s"),
        out_type=[
            jax.ShapeDtypeStruct((B, D_H), jnp.float32),
            jax.ShapeDtypeStruct((B,), jnp.int32),
        ],
        scratch_types=[
            pltpu.VMEM((_BPW,), jnp.int32),
            pltpu.VMEM((_BPW, D_H), jnp.float32),
            pltpu.VMEM((_BPW,), jnp.int32),
            pltpu.SemaphoreType.DMA,
            pltpu.SemaphoreType.DMA,
        ],
    )(_gather_body)


def _head_body(x_ref, h_ref, c_ref, mem_ref, wi_ref, wh_ref, bl_ref,
               wa_ref, ba_ref, wpi_ref, bpi_ref, wv_ref, bv_ref,
               a_ref, p_ref, v_ref, e_ref, ho_ref, co_ref):
    x = x_ref[...]
    hh = h_ref[...]
    cc = c_ref[...]
    dn = (((1,), (1,)), ((), ()))
    preact = (lax.dot_general(x, wi_ref[...], dn,
                              preferred_element_type=jnp.float32)
              + lax.dot_general(hh, wh_ref[...], dn,
                                preferred_element_type=jnp.float32)
              + bl_ref[...])
    gates = jax.nn.sigmoid(preact[:, :N_GATES * D_H])
    f_t = gates[:, :D_H]
    i_t = gates[:, D_H:2 * D_H]
    o_t = gates[:, 2 * D_H:3 * D_H]
    r_t = gates[:, 3 * D_H:4 * D_H]
    c_tilde = jnp.tanh(preact[:, N_GATES * D_H:])
    m_t = jnp.tanh(mem_ref[...])
    c_t = f_t * cc + i_t * c_tilde + r_t * m_t
    h_t = o_t * jnp.tanh(c_t)
    a_hid = jnp.maximum(
        lax.dot_general(h_t, wa_ref[...], dn,
                        preferred_element_type=jnp.float32) + ba_ref[...], 0.0)
    logits = (lax.dot_general(a_hid, wpi_ref[...], dn,
                              preferred_element_type=jnp.float32)
              + bpi_ref[...])
    v_ref[...] = (jnp.sum(a_hid * wv_ref[...], axis=1, keepdims=True)
                  + bv_ref[...])
    m = jnp.max(logits, axis=1, keepdims=True)
    e = jnp.exp(logits - m)
    pi = e / jnp.sum(e, axis=1, keepdims=True)
    pmax = jnp.max(pi, axis=1, keepdims=True)
    col = lax.broadcasted_iota(jnp.int32, (B, D_OUT), 1)
    a_ref[...] = jnp.min(jnp.where(pi == pmax, col, D_OUT),
                         axis=1, keepdims=True)
    p_ref[...] = jnp.log(pmax + 1e-12)
    e_ref[...] = -jnp.sum(pi * jnp.log(pi + 1e-12), axis=1, keepdims=True)
    ho_ref[...] = h_t
    co_ref[...] = c_t


_head_call = pl.pallas_call(
    _head_body,
    out_shape=(
        jax.ShapeDtypeStruct((B, 1), jnp.int32),
        jax.ShapeDtypeStruct((B, 1), jnp.float32),
        jax.ShapeDtypeStruct((B, 1), jnp.float32),
        jax.ShapeDtypeStruct((B, 1), jnp.float32),
        jax.ShapeDtypeStruct((B, D_H), jnp.float32),
        jax.ShapeDtypeStruct((B, D_H), jnp.float32),
    ),
)


def kernel(obs_bar_reward, barcode_tensor, barcode_id, h, c, dnd_keys,
           dnd_vals, key_id_map, W_i2h, b_i2h, W_h2h, b_h2h,
           W_a2c_h, b_a2c_h, W_pi, b_pi, W_v, b_v):
    best = _sim_call(barcode_tensor, dnd_keys).reshape(B)
    mem, predicted_barcode = _sc_gather_call()(best, dnd_vals, key_id_map)

    a_t, prob_a_t, v_t, entropy, h_t, c_t = _head_call(
        obs_bar_reward, h, c, mem,
        W_i2h, W_h2h, (b_i2h + b_h2h).reshape(1, -1),
        W_a2c_h, b_a2c_h.reshape(1, -1),
        W_pi, b_pi.reshape(1, -1), W_v, b_v.reshape(1, -1))
    return (a_t.reshape(B), predicted_barcode, prob_a_t.reshape(B), v_t,
            entropy.reshape(B), h_t, c_t)
